# Initial kernel scaffold; baseline (speedup 1.0000x reference)
import jax
import jax.numpy as jnp
from jax.experimental import pallas as pl


def kernel(img_idx, juncs_pred, meta):
    raise NotImplementedError("write your pallas kernel here")


# speedup vs baseline: 7.4396x; 7.4396x over previous
import functools

import jax
import jax.numpy as jnp
from jax import lax
from jax.experimental import pallas as pl
from jax.experimental.pallas import tpu as pltpu
from jax.experimental.pallas import tpu_sc as plsc

N = 1024
P = N * (N - 1) // 2
NC = 2
NS = 16
NW = NC * NS
PW = P // NW
L = 16
TRIP = PW // L + 97 + 16


def _off_next(iv):
    return (iv + 1) * (N - 1) - lax.shift_right_logical((iv + 1) * iv, 1)


def _pairs_body(juncs_hbm, lines_hbm, jidx_hbm, table_v, lines_v, jidx_v):
    wid = lax.axis_index("s") * NC + lax.axis_index("c")
    p0 = wid * PW
    p1 = p0 + PW

    pltpu.sync_copy(juncs_hbm, table_v)

    lane = lax.broadcasted_iota(jnp.int32, (L,), 0)
    zero_v = jnp.zeros((L,), jnp.int32)
    one_v = jnp.ones((L,), jnp.int32)

    def _off(i):
        return i * (N - 1) - lax.shift_right_logical(i * (i - 1), 1)

    def bisect_body(_, c):
        lo, hi = c
        mid = lax.shift_right_logical(lo + hi + 1, 1)
        pred = _off(mid) <= p0
        return jnp.where(pred, mid, lo), jnp.where(pred, hi, mid - 1)

    i0, _ = lax.fori_loop(0, 10, bisect_body,
                          (jnp.int32(0), jnp.int32(N - 1)))
    j0 = i0 + 1 + (p0 - _off(i0))

    def body(_, c):
        p, i, j, q = c
        cnt = jnp.maximum(
            jnp.minimum(jnp.minimum(jnp.int32(L), N - j), p1 - p),
            jnp.int32(0))
        msk = lane < cnt
        jv = jnp.minimum(j + lane, jnp.int32(N - 1))
        iv = jnp.full((L,), jnp.minimum(i, jnp.int32(N - 1)), jnp.int32)
        xj = plsc.load_gather(table_v, [jv * 2], mask=msk)
        yj = plsc.load_gather(table_v, [jv * 2 + 1], mask=msk)
        xi = plsc.load_gather(table_v, [iv * 2], mask=msk)
        yi = plsc.load_gather(table_v, [iv * 2 + 1], mask=msk)
        r = jnp.minimum(q + lane, jnp.int32(PW - 1))
        r4 = r * 4
        r2 = r * 2
        plsc.store_scatter(lines_v, [r4], xi, mask=msk)
        plsc.store_scatter(lines_v, [r4 + 1], yi, mask=msk)
        plsc.store_scatter(lines_v, [r4 + 2], xj, mask=msk)
        plsc.store_scatter(lines_v, [r4 + 3], yj, mask=msk)
        plsc.store_scatter(jidx_v, [r2], iv, mask=msk)
        plsc.store_scatter(jidx_v, [r2 + 1], j + lane, mask=msk)
        p = p + cnt
        q = q + cnt
        j = j + cnt
        row_done = jnp.logical_and(j >= N, p < p1)
        i = jnp.where(row_done, i + 1, i)
        j = jnp.where(row_done, i + 1, j)
        return p, i, j, q

    lax.fori_loop(0, TRIP, body, (p0, i0, j0, jnp.int32(0)))

    pltpu.sync_copy(lines_v, lines_hbm.at[pl.ds(p0 * 4, PW * 4)])
    pltpu.sync_copy(jidx_v, jidx_hbm.at[pl.ds(p0 * 2, PW * 2)])


@jax.jit
def _pairs(juncs_pred):
    mesh = plsc.VectorSubcoreMesh(core_axis_name="c", subcore_axis_name="s")
    k = functools.partial(
        pl.kernel,
        mesh=mesh,
        out_type=[
            jax.ShapeDtypeStruct((P * 4,), jnp.float32),
            jax.ShapeDtypeStruct((P * 2,), jnp.int32),
        ],
        scratch_types=[
            pltpu.VMEM((N * 2,), jnp.float32),
            pltpu.VMEM((PW * 4,), jnp.float32),
            pltpu.VMEM((PW * 2,), jnp.int32),
        ],
        compiler_params=pltpu.CompilerParams(needs_layout_passes=False),
    )(_pairs_body)
    lines_flat, jidx_flat = k(juncs_pred.reshape(-1))
    return lines_flat.reshape(P, 4), jidx_flat.reshape(P, 2)


def kernel(img_idx, juncs_pred, meta):
    lines_pred, jidx = _pairs(juncs_pred)
    labels = jnp.ones((P,), dtype=jnp.int32)
    label_scores = jnp.ones((P,), dtype=jnp.float32)
    scores = jnp.zeros((P, 2), dtype=jnp.float32).at[:, 1].set(1.0)
    return (lines_pred, labels, label_scores, jidx, scores)


# speedup vs baseline: 133.3809x; 17.9284x over previous
import functools

import jax
import jax.numpy as jnp
from jax import lax
from jax.experimental import pallas as pl
from jax.experimental.pallas import tpu as pltpu
from jax.experimental.pallas import tpu_sc as plsc

N = 1024
P = N * (N - 1) // 2
NB = P // 128
NC = 2
NS = 16
NW = NC * NS
NBW = 128
L = 16
TRIP = 1120


def _pairs_body(juncs_hbm, lines_hbm, jidx_hbm, table_v, lines_v, jidx_v):
    wid = lax.axis_index("s") * NC + lax.axis_index("c")
    big = wid < 28
    b0 = jnp.where(big, wid * 128, 28 + wid * 127)
    p0 = b0 * 128
    p1 = p0 + jnp.where(big, 128 * 128, 127 * 128)

    pltpu.sync_copy(juncs_hbm, table_v)

    lane = lax.broadcasted_iota(jnp.int32, (L,), 0)
    zero_v = jnp.zeros((L,), jnp.int32)
    one_v = jnp.ones((L,), jnp.int32)

    def _off(i):
        return i * (N - 1) - lax.shift_right_logical(i * (i - 1), 1)

    def bisect_body(_, c):
        lo, hi = c
        mid = lax.shift_right_logical(lo + hi + 1, 1)
        pred = _off(mid) <= p0
        return jnp.where(pred, mid, lo), jnp.where(pred, hi, mid - 1)

    i0, _ = lax.fori_loop(0, 10, bisect_body,
                          (jnp.int32(0), jnp.int32(N - 1)))
    j0 = i0 + 1 + (p0 - _off(i0))

    def body(_, c):
        p, i, j, q = c
        cnt = jnp.maximum(
            jnp.minimum(jnp.minimum(jnp.int32(L), N - j), p1 - p),
            jnp.int32(0))
        msk = lane < cnt
        jv = jnp.minimum(j + lane, jnp.int32(N - 1))
        iv = jnp.full((L,), jnp.minimum(i, jnp.int32(N - 1)), jnp.int32)
        xj = plsc.load_gather(table_v, [jv * 2], mask=msk)
        yj = plsc.load_gather(table_v, [jv * 2 + 1], mask=msk)
        xi = plsc.load_gather(table_v, [iv * 2], mask=msk)
        yi = plsc.load_gather(table_v, [iv * 2 + 1], mask=msk)
        r = jnp.minimum(q + lane, jnp.int32(NBW * 128 - 1))
        blk = lax.shift_right_logical(r, 7)
        low = jnp.bitwise_and(r, jnp.int32(127))
        b4 = lax.shift_left(blk, 9) + low
        b2 = lax.shift_left(blk, 8) + low
        plsc.store_scatter(lines_v, [b4], xi, mask=msk)
        plsc.store_scatter(lines_v, [b4 + 128], yi, mask=msk)
        plsc.store_scatter(lines_v, [b4 + 256], xj, mask=msk)
        plsc.store_scatter(lines_v, [b4 + 384], yj, mask=msk)
        plsc.store_scatter(jidx_v, [b2], iv, mask=msk)
        plsc.store_scatter(jidx_v, [b2 + 128], j + lane, mask=msk)
        p = p + cnt
        q = q + cnt
        j = j + cnt
        row_done = jnp.logical_and(j >= N, p < p1)
        i = jnp.where(row_done, i + 1, i)
        j = jnp.where(row_done, i + 1, j)
        return p, i, j, q

    lax.fori_loop(0, TRIP, body, (p0, i0, j0, jnp.int32(0)))

    @pl.when(big)
    def _():
        pltpu.sync_copy(lines_v,
                        lines_hbm.at[pl.ds(b0 * 512, 128 * 512)])
        pltpu.sync_copy(jidx_v,
                        jidx_hbm.at[pl.ds(b0 * 256, 128 * 256)])

    @pl.when(jnp.logical_not(big))
    def _():
        pltpu.sync_copy(lines_v.at[pl.ds(0, 127 * 512)],
                        lines_hbm.at[pl.ds(b0 * 512, 127 * 512)])
        pltpu.sync_copy(jidx_v.at[pl.ds(0, 127 * 256)],
                        jidx_hbm.at[pl.ds(b0 * 256, 127 * 256)])


@jax.jit
def _pairs(juncs_pred):
    mesh = plsc.VectorSubcoreMesh(core_axis_name="c", subcore_axis_name="s")
    k = functools.partial(
        pl.kernel,
        mesh=mesh,
        out_type=[
            jax.ShapeDtypeStruct((P * 4,), jnp.float32),
            jax.ShapeDtypeStruct((P * 2,), jnp.int32),
        ],
        scratch_types=[
            pltpu.VMEM((N * 2,), jnp.float32),
            pltpu.VMEM((NBW * 512,), jnp.float32),
            pltpu.VMEM((NBW * 256,), jnp.int32),
        ],
        compiler_params=pltpu.CompilerParams(needs_layout_passes=False),
    )(_pairs_body)
    lines_flat, jidx_flat = k(juncs_pred.reshape(-1))
    lines_pred = (lines_flat.reshape(NB, 4, 128)
                  .transpose(0, 2, 1).reshape(P, 4))
    jidx = (jidx_flat.reshape(NB, 2, 128)
            .transpose(0, 2, 1).reshape(P, 2))
    return lines_pred, jidx


def kernel(img_idx, juncs_pred, meta):
    lines_pred, jidx = _pairs(juncs_pred)
    labels = jnp.ones((P,), dtype=jnp.int32)
    label_scores = jnp.ones((P,), dtype=jnp.float32)
    scores = jnp.zeros((P, 2), dtype=jnp.float32).at[:, 1].set(1.0)
    return (lines_pred, labels, label_scores, jidx, scores)


# speedup vs baseline: 137.0987x; 1.0279x over previous
import functools

import jax
import jax.numpy as jnp
from jax import lax
from jax.experimental import pallas as pl
from jax.experimental.pallas import tpu as pltpu
from jax.experimental.pallas import tpu_sc as plsc

N = 1024
P = N * (N - 1) // 2
NB = P // 128
NC = 2
NS = 16
NW = NC * NS
NBW = 128
L = 16
TRIP = 1120


def _pairs_body(juncs_hbm, lines_hbm, jidx_hbm, table_v, lines_v, jidx_v):
    wid = lax.axis_index("s") * NC + lax.axis_index("c")
    big = wid < 28
    b0 = jnp.where(big, wid * 128, 28 + wid * 127)
    p0 = b0 * 128
    p1 = p0 + jnp.where(big, 128 * 128, 127 * 128)

    pltpu.sync_copy(juncs_hbm, table_v)

    lane = lax.broadcasted_iota(jnp.int32, (L,), 0)
    zero_v = jnp.zeros((L,), jnp.int32)
    one_v = jnp.ones((L,), jnp.int32)

    def _off(i):
        return i * (N - 1) - lax.shift_right_logical(i * (i - 1), 1)

    def bisect_body(_, c):
        lo, hi = c
        mid = lax.shift_right_logical(lo + hi + 1, 1)
        pred = _off(mid) <= p0
        return jnp.where(pred, mid, lo), jnp.where(pred, hi, mid - 1)

    i0, _ = lax.fori_loop(0, 10, bisect_body,
                          (jnp.int32(0), jnp.int32(N - 1)))
    j0 = i0 + 1 + (p0 - _off(i0))

    def _row_vecs(i):
        iv = jnp.full((L,), i, jnp.int32)
        xi = plsc.load_gather(table_v, [iv * 2])
        yi = plsc.load_gather(table_v, [iv * 2 + 1])
        return iv, xi, yi

    def body(_, c):
        p, i, j, q, iv, xi, yi = c
        cnt = jnp.minimum(jnp.minimum(jnp.int32(L), N - j), p1 - p)
        msk = lane < cnt
        jv = j + lane
        xj = plsc.load_gather(table_v, [jv * 2], mask=msk)
        yj = plsc.load_gather(table_v, [jv * 2 + 1], mask=msk)
        r = q + lane
        blk = lax.shift_right_logical(r, 7)
        low = jnp.bitwise_and(r, jnp.int32(127))
        b4 = lax.shift_left(blk, 9) + low
        b2 = lax.shift_left(blk, 8) + low
        plsc.store_scatter(lines_v, [b4], xi, mask=msk)
        plsc.store_scatter(lines_v, [b4 + 128], yi, mask=msk)
        plsc.store_scatter(lines_v, [b4 + 256], xj, mask=msk)
        plsc.store_scatter(lines_v, [b4 + 384], yj, mask=msk)
        plsc.store_scatter(jidx_v, [b2], iv, mask=msk)
        plsc.store_scatter(jidx_v, [b2 + 128], jv, mask=msk)
        p = p + cnt
        q = q + cnt
        j = j + cnt
        row_done = jnp.logical_and(j >= N, p < p1)
        i = jnp.where(row_done, i + 1, i)
        j = jnp.where(row_done, i + 1, j)
        iv, xi, yi = lax.cond(row_done, _row_vecs,
                              lambda _: (iv, xi, yi), i)
        return p, i, j, q, iv, xi, yi

    iv0, xi0, yi0 = _row_vecs(i0)
    lax.fori_loop(0, TRIP, body,
                  (p0, i0, j0, jnp.int32(0), iv0, xi0, yi0))

    @pl.when(big)
    def _():
        pltpu.sync_copy(lines_v,
                        lines_hbm.at[pl.ds(b0 * 512, 128 * 512)])
        pltpu.sync_copy(jidx_v,
                        jidx_hbm.at[pl.ds(b0 * 256, 128 * 256)])

    @pl.when(jnp.logical_not(big))
    def _():
        pltpu.sync_copy(lines_v.at[pl.ds(0, 127 * 512)],
                        lines_hbm.at[pl.ds(b0 * 512, 127 * 512)])
        pltpu.sync_copy(jidx_v.at[pl.ds(0, 127 * 256)],
                        jidx_hbm.at[pl.ds(b0 * 256, 127 * 256)])


@jax.jit
def _pairs(juncs_pred):
    mesh = plsc.VectorSubcoreMesh(core_axis_name="c", subcore_axis_name="s")
    k = functools.partial(
        pl.kernel,
        mesh=mesh,
        out_type=[
            jax.ShapeDtypeStruct((P * 4,), jnp.float32),
            jax.ShapeDtypeStruct((P * 2,), jnp.int32),
        ],
        scratch_types=[
            pltpu.VMEM((N * 2,), jnp.float32),
            pltpu.VMEM((NBW * 512,), jnp.float32),
            pltpu.VMEM((NBW * 256,), jnp.int32),
        ],
        compiler_params=pltpu.CompilerParams(needs_layout_passes=False),
    )(_pairs_body)
    lines_flat, jidx_flat = k(juncs_pred.reshape(-1))
    lines_pred = (lines_flat.reshape(NB, 4, 128)
                  .transpose(0, 2, 1).reshape(P, 4))
    jidx = (jidx_flat.reshape(NB, 2, 128)
            .transpose(0, 2, 1).reshape(P, 2))
    return lines_pred, jidx


def kernel(img_idx, juncs_pred, meta):
    lines_pred, jidx = _pairs(juncs_pred)
    labels = jnp.ones((P,), dtype=jnp.int32)
    label_scores = jnp.ones((P,), dtype=jnp.float32)
    scores = jnp.broadcast_to(jnp.array([0.0, 1.0], dtype=jnp.float32),
                              (P, 2))
    return (lines_pred, labels, label_scores, jidx, scores)
